# baseline (speedup 1.0000x reference)
import functools

import jax
import jax.numpy as jnp
from jax import lax
from jax.experimental import pallas as pl
from jax.experimental.pallas import tpu as pltpu
from jax.experimental.pallas import tpu_sc as plsc

_NUM_CORES = 2
_NUM_SUBCORES = 16
_NUM_WORKERS = _NUM_CORES * _NUM_SUBCORES
_LANES = 16

_B = 16384
_L = 50
_D = 64
_DP = 2 * _D
_DV = _D // _LANES

_SAMPLES_PER_W = _B // _NUM_WORKERS
_CHUNK_SAMPLES = 16
_CHUNK_ROWS = _CHUNK_SAMPLES * _L
_GATHER_ROWS = 80
_GATHERS_PER_CHUNK = _CHUNK_ROWS // _GATHER_ROWS
_CHUNKS_PER_W = _SAMPLES_PER_W // _CHUNK_SAMPLES


def _sc_pool(table2, idx_pair, off_col):
    mesh = plsc.VectorSubcoreMesh(
        core_axis_name="c", subcore_axis_name="s",
        num_cores=_NUM_CORES, num_subcores=_NUM_SUBCORES)

    @functools.partial(
        pl.kernel,
        out_type=jax.ShapeDtypeStruct((_B, _D), jnp.float32),
        mesh=mesh,
        scratch_types=[
            pltpu.VMEM((_CHUNK_ROWS,), jnp.int32),
            pltpu.VMEM((_CHUNK_ROWS + _LANES,), jnp.int32),
            pltpu.VMEM((_CHUNK_ROWS, _DP), jnp.float32),
            pltpu.VMEM((_CHUNK_SAMPLES, _D), jnp.float32),
            pltpu.SemaphoreType.DMA,
        ],
        compiler_params=pltpu.CompilerParams(use_tc_tiling_on_sc=True),
    )
    def sc_kernel(table_hbm, idx_hbm, off_hbm, out_hbm,
                  idx_v, off_v, rows_v, pooled_v, sem):
        wid = lax.axis_index("s") * _NUM_CORES + lax.axis_index("c")
        w_row0 = wid * (_SAMPLES_PER_W * _L)
        w_samp0 = wid * _SAMPLES_PER_W

        def chunk_body(t, carry):
            row0 = w_row0 + t * _CHUNK_ROWS
            pltpu.sync_copy(idx_hbm.at[pl.ds(row0, _CHUNK_ROWS)], idx_v)
            pltpu.sync_copy(off_hbm.at[pl.ds(row0, _CHUNK_ROWS)],
                            off_v.at[pl.ds(0, _CHUNK_ROWS)])
            copies = []
            for g in range(_GATHERS_PER_CHUNK):
                sl = pl.ds(g * _GATHER_ROWS, _GATHER_ROWS)
                copies.append(pltpu.async_copy(
                    table_hbm.at[idx_v.at[sl]], rows_v.at[sl], sem))
            for c in copies:
                c.wait()

            def sample_body(s, carry2):
                accs = [jnp.zeros((_LANES,), jnp.float32) for _ in range(_DV)]
                base = s * _L
                off_vecs = [off_v[pl.ds(base + m * _LANES, _LANES)]
                            for m in range(-(-_L // _LANES))]
                for j in range(_L):
                    r = base + j
                    o = off_vecs[j // _LANES][j % _LANES]
                    for k in range(_DV):
                        accs[k] = accs[k] + rows_v[r, pl.ds(o + k * _LANES, _LANES)]
                for k in range(_DV):
                    pooled_v[s, pl.ds(k * _LANES, _LANES)] = accs[k]
                return carry2

            lax.fori_loop(0, _CHUNK_SAMPLES, sample_body, 0)
            pltpu.sync_copy(
                pooled_v,
                out_hbm.at[pl.ds(w_samp0 + t * _CHUNK_SAMPLES, _CHUNK_SAMPLES)])
            return carry

        lax.fori_loop(0, _CHUNKS_PER_W, chunk_body, 0)

    return sc_kernel(table2, idx_pair, off_col)


def _mlp_body(x_ref, w1_ref, b1_ref, w2_ref, b2_ref, o_ref):
    x = x_ref[...] * (1.0 / _L)
    h = jnp.dot(x, w1_ref[...], preferred_element_type=jnp.float32)
    h = jnp.maximum(h + b1_ref[...], 0.0)
    o = jnp.dot(h, w2_ref[...], preferred_element_type=jnp.float32)
    o_ref[...] = o + b2_ref[...]


def _tc_mlp(pooled, w1t, b1, w2t, b2):
    bm = 2048
    h1 = w1t.shape[1]
    h2 = w2t.shape[1]
    return pl.pallas_call(
        _mlp_body,
        grid=(_B // bm,),
        in_specs=[
            pl.BlockSpec((bm, _D), lambda i: (i, 0)),
            pl.BlockSpec((_D, h1), lambda i: (0, 0)),
            pl.BlockSpec((1, h1), lambda i: (0, 0)),
            pl.BlockSpec((h1, h2), lambda i: (0, 0)),
            pl.BlockSpec((1, h2), lambda i: (0, 0)),
        ],
        out_specs=pl.BlockSpec((bm, h2), lambda i: (i, 0)),
        out_shape=jax.ShapeDtypeStruct((_B, h2), jnp.float32),
    )(pooled, w1t, b1, w2t, b2)


def kernel(arg0_1, arg1_1, arg2_1, arg3_1, arg4_1, arg5_1):
    table2 = arg0_1.reshape(-1, _DP)
    idx_flat = arg1_1.reshape(-1)
    idx_pair = idx_flat >> 1
    off_col = (idx_flat & 1) * _D
    pooled = _sc_pool(table2, idx_pair, off_col)
    w1t = arg2_1.T
    w2t = arg4_1.T
    b1 = arg3_1.reshape(1, -1)
    b2 = arg5_1.reshape(1, -1)
    out = _tc_mlp(pooled, w1t, b1, w2t, b2)
    return (out,)


# speedup vs baseline: 1.3801x; 1.3801x over previous
import functools

import jax
import jax.numpy as jnp
from jax import lax
from jax.experimental import pallas as pl
from jax.experimental.pallas import tpu as pltpu
from jax.experimental.pallas import tpu_sc as plsc

_NUM_CORES = 2
_NUM_SUBCORES = 16
_NUM_WORKERS = _NUM_CORES * _NUM_SUBCORES
_LANES = 16

_B = 16384
_L = 50
_D = 64
_DP = 2 * _D
_DV = _D // _LANES

_SAMPLES_PER_W = _B // _NUM_WORKERS
_CHUNK_SAMPLES = 16
_CHUNK_ROWS = _CHUNK_SAMPLES * _L
_GATHER_ROWS = 80
_GATHERS_PER_CHUNK = _CHUNK_ROWS // _GATHER_ROWS
_CHUNKS_PER_W = _SAMPLES_PER_W // _CHUNK_SAMPLES


def _sc_pool(table2, idx_pair, off_col):
    mesh = plsc.VectorSubcoreMesh(
        core_axis_name="c", subcore_axis_name="s",
        num_cores=_NUM_CORES, num_subcores=_NUM_SUBCORES)

    @functools.partial(
        pl.kernel,
        out_type=jax.ShapeDtypeStruct((_B, _D), jnp.float32),
        mesh=mesh,
        scratch_types=[
            pltpu.VMEM((_CHUNK_ROWS,), jnp.int32),
            pltpu.VMEM((_CHUNK_ROWS + _LANES,), jnp.int32),
            pltpu.VMEM((_CHUNK_ROWS, _DP), jnp.float32),
            pltpu.VMEM((_CHUNK_SAMPLES, _D), jnp.float32),
            pltpu.SemaphoreType.DMA,
        ],
        compiler_params=pltpu.CompilerParams(use_tc_tiling_on_sc=True),
    )
    def sc_kernel(table_hbm, idx_hbm, off_hbm, out_hbm,
                  idx_v, off_v, rows_v, pooled_v, sem):
        wid = lax.axis_index("s") * _NUM_CORES + lax.axis_index("c")
        w_row0 = wid * (_SAMPLES_PER_W * _L)
        w_samp0 = wid * _SAMPLES_PER_W

        def chunk_body(t, carry):
            row0 = w_row0 + t * _CHUNK_ROWS
            pltpu.sync_copy(idx_hbm.at[pl.ds(row0, _CHUNK_ROWS)], idx_v)
            pltpu.sync_copy(off_hbm.at[pl.ds(row0, _CHUNK_ROWS)],
                            off_v.at[pl.ds(0, _CHUNK_ROWS)])
            copies = []
            for g in range(_GATHERS_PER_CHUNK):
                sl = pl.ds(g * _GATHER_ROWS, _GATHER_ROWS)
                copies.append(pltpu.async_copy(
                    table_hbm.at[idx_v.at[sl]], rows_v.at[sl], sem))
            for c in copies:
                c.wait()

            def sample_body(s, carry2):
                accs = [jnp.zeros((_LANES,), jnp.float32) for _ in range(_DV)]
                base = s * _L
                off_vecs = [off_v[pl.ds(base + m * _LANES, _LANES)]
                            for m in range(-(-_L // _LANES))]
                for j in range(_L):
                    r = base + j
                    o = off_vecs[j // _LANES][j % _LANES]
                    for k in range(_DV):
                        accs[k] = accs[k] + rows_v[r, pl.ds(o + k * _LANES, _LANES)]
                for k in range(_DV):
                    pooled_v[s, pl.ds(k * _LANES, _LANES)] = accs[k]
                return carry2

            lax.fori_loop(0, _CHUNK_SAMPLES, sample_body, 0)
            pltpu.sync_copy(
                pooled_v,
                out_hbm.at[pl.ds(w_samp0 + t * _CHUNK_SAMPLES, _CHUNK_SAMPLES)])
            return carry

        lax.fori_loop(0, _CHUNKS_PER_W, chunk_body, 0)

    return sc_kernel(table2, idx_pair, off_col)


def _pack_body(a_ref, b_ref, o_ref):
    o_ref[...] = jnp.concatenate(
        [jnp.transpose(a_ref[...]), jnp.transpose(b_ref[...])], axis=1)


_PACK_BM = 2048
_HALF = _PACK_BM * 245


def _tc_pack(table_t):
    nb = _HALF // _PACK_BM
    last = -(-table_t.shape[1] // _PACK_BM) - 1
    return pl.pallas_call(
        _pack_body,
        grid=(nb,),
        in_specs=[
            pl.BlockSpec((_D, _PACK_BM), lambda i: (0, i)),
            pl.BlockSpec((_D, _PACK_BM), lambda i: (0, jnp.minimum(i + nb, last))),
        ],
        out_specs=pl.BlockSpec((_PACK_BM, _DP), lambda i: (i, 0)),
        out_shape=jax.ShapeDtypeStruct((_HALF, _DP), jnp.float32),
    )(table_t, table_t)


def _mlp_body(x_ref, w1_ref, b1_ref, w2_ref, b2_ref, o_ref):
    x = x_ref[...] * (1.0 / _L)
    h = jnp.dot(x, w1_ref[...], preferred_element_type=jnp.float32)
    h = jnp.maximum(h + b1_ref[...], 0.0)
    o = jnp.dot(h, w2_ref[...], preferred_element_type=jnp.float32)
    o_ref[...] = o + b2_ref[...]


def _tc_mlp(pooled, w1t, b1, w2t, b2):
    bm = 2048
    h1 = w1t.shape[1]
    h2 = w2t.shape[1]
    return pl.pallas_call(
        _mlp_body,
        grid=(_B // bm,),
        in_specs=[
            pl.BlockSpec((bm, _D), lambda i: (i, 0)),
            pl.BlockSpec((_D, h1), lambda i: (0, 0)),
            pl.BlockSpec((1, h1), lambda i: (0, 0)),
            pl.BlockSpec((h1, h2), lambda i: (0, 0)),
            pl.BlockSpec((1, h2), lambda i: (0, 0)),
        ],
        out_specs=pl.BlockSpec((bm, h2), lambda i: (i, 0)),
        out_shape=jax.ShapeDtypeStruct((_B, h2), jnp.float32),
    )(pooled, w1t, b1, w2t, b2)


def kernel(arg0_1, arg1_1, arg2_1, arg3_1, arg4_1, arg5_1):
    table2 = _tc_pack(arg0_1.T)
    idx_flat = arg1_1.reshape(-1)
    hi = idx_flat >= _HALF
    idx_pair = jnp.where(hi, idx_flat - _HALF, idx_flat)
    off_col = hi.astype(jnp.int32) * _D
    pooled = _sc_pool(table2, idx_pair, off_col)
    w1t = arg2_1.T
    w2t = arg4_1.T
    b1 = arg3_1.reshape(1, -1)
    b2 = arg5_1.reshape(1, -1)
    out = _tc_mlp(pooled, w1t, b1, w2t, b2)
    return (out,)


# speedup vs baseline: 1.7162x; 1.2435x over previous
import functools

import jax
import jax.numpy as jnp
from jax import lax
from jax.experimental import pallas as pl
from jax.experimental.pallas import tpu as pltpu
from jax.experimental.pallas import tpu_sc as plsc

_NUM_CORES = 2
_NUM_SUBCORES = 16
_NUM_WORKERS = _NUM_CORES * _NUM_SUBCORES
_LANES = 16

_B = 16384
_L = 50
_D = 64
_DP = 2 * _D
_DV = _D // _LANES

_SAMPLES_PER_W = _B // _NUM_WORKERS
_CHUNK_SAMPLES = 8
_CHUNK_ROWS = _CHUNK_SAMPLES * _L
_GATHER_ROWS = 80
_GATHERS_PER_CHUNK = _CHUNK_ROWS // _GATHER_ROWS
_CHUNKS_PER_W = _SAMPLES_PER_W // _CHUNK_SAMPLES


def _sc_pool(table2, idx_pair, off_col):
    mesh = plsc.VectorSubcoreMesh(
        core_axis_name="c", subcore_axis_name="s",
        num_cores=_NUM_CORES, num_subcores=_NUM_SUBCORES)

    nbuf = 2

    @functools.partial(
        pl.kernel,
        out_type=jax.ShapeDtypeStruct((_B, _D), jnp.float32),
        mesh=mesh,
        scratch_types=[
            pltpu.VMEM((nbuf * _CHUNK_ROWS,), jnp.int32),
            pltpu.VMEM((nbuf * (_CHUNK_ROWS + _LANES),), jnp.int32),
            pltpu.VMEM((nbuf, _CHUNK_ROWS, _DP), jnp.float32),
            pltpu.VMEM((nbuf * _CHUNK_SAMPLES, _D), jnp.float32),
            pltpu.SemaphoreType.DMA,
            pltpu.SemaphoreType.DMA,
            pltpu.SemaphoreType.DMA,
            pltpu.SemaphoreType.DMA,
            pltpu.SemaphoreType.DMA,
            pltpu.SemaphoreType.DMA,
        ],
        compiler_params=pltpu.CompilerParams(use_tc_tiling_on_sc=True),
    )
    def sc_kernel(table_hbm, idx_hbm, off_hbm, out_hbm,
                  idx_v, off_v, rows_v, pooled_v,
                  semi0, semi1, semg0, semg1, semo0, semo1):
        semi = (semi0, semi1)
        semg = (semg0, semg1)
        semo = (semo0, semo1)
        wid = lax.axis_index("s") * _NUM_CORES + lax.axis_index("c")
        w_row0 = wid * (_SAMPLES_PER_W * _L)
        w_samp0 = wid * _SAMPLES_PER_W

        def idx_copies(c, b):
            base = w_row0 + c * _CHUNK_ROWS
            return (
                pltpu.make_async_copy(
                    idx_hbm.at[pl.ds(base, _CHUNK_ROWS)],
                    idx_v.at[pl.ds(b * _CHUNK_ROWS, _CHUNK_ROWS)], semi[b]),
                pltpu.make_async_copy(
                    off_hbm.at[pl.ds(base, _CHUNK_ROWS)],
                    off_v.at[pl.ds(b * (_CHUNK_ROWS + _LANES), _CHUNK_ROWS)],
                    semi[b]),
            )

        def gather_copies(b):
            out = []
            for g in range(_GATHERS_PER_CHUNK):
                sl = pl.ds(g * _GATHER_ROWS, _GATHER_ROWS)
                out.append(pltpu.make_async_copy(
                    table_hbm.at[idx_v.at[pl.ds(
                        b * _CHUNK_ROWS + g * _GATHER_ROWS, _GATHER_ROWS)]],
                    rows_v.at[b].at[sl], semg[b]))
            return out

        def out_copy(c, b):
            return pltpu.make_async_copy(
                pooled_v.at[pl.ds(b * _CHUNK_SAMPLES, _CHUNK_SAMPLES)],
                out_hbm.at[pl.ds(w_samp0 + c * _CHUNK_SAMPLES, _CHUNK_SAMPLES)],
                semo[b])

        def accumulate(b):
            def sample_body(s, carry2):
                accs = [jnp.zeros((_LANES,), jnp.float32) for _ in range(_DV)]
                base = s * _L
                ob = b * (_CHUNK_ROWS + _LANES)
                off_vecs = [off_v[pl.ds(ob + base + m * _LANES, _LANES)]
                            for m in range(-(-_L // _LANES))]
                for j in range(_L):
                    r = base + j
                    o = off_vecs[j // _LANES][j % _LANES]
                    for k in range(_DV):
                        accs[k] = accs[k] + rows_v[b, r, pl.ds(o + k * _LANES, _LANES)]
                for k in range(_DV):
                    pooled_v[b * _CHUNK_SAMPLES + s,
                             pl.ds(k * _LANES, _LANES)] = accs[k]
                return carry2

            lax.fori_loop(0, _CHUNK_SAMPLES, sample_body, 0)

        for cp in idx_copies(0, 0):
            cp.start()
        for cp in idx_copies(1, 1):
            cp.start()
        for cp in idx_copies(0, 0):
            cp.wait()
        for cp in gather_copies(0):
            cp.start()

        def pair_body(it, carry):
            for b in range(nbuf):
                c = it * nbuf + b
                nb = 1 - b

                @pl.when(c < _CHUNKS_PER_W - 1)
                def _():
                    for cp in idx_copies(c + 1, nb):
                        cp.wait()
                    for cp in gather_copies(nb):
                        cp.start()

                for cp in gather_copies(b):
                    cp.wait()

                @pl.when(c >= 2)
                def _():
                    out_copy(c - 2, b).wait()

                accumulate(b)

                @pl.when(c + 2 < _CHUNKS_PER_W)
                def _():
                    for cp in idx_copies(c + 2, b):
                        cp.start()

                out_copy(c, b).start()
            return carry

        lax.fori_loop(0, _CHUNKS_PER_W // nbuf, pair_body, 0)
        out_copy(_CHUNKS_PER_W - 2, 0).wait()
        out_copy(_CHUNKS_PER_W - 1, 1).wait()

    return sc_kernel(table2, idx_pair, off_col)


def _pack_body(a_ref, b_ref, o_ref):
    o_ref[...] = jnp.concatenate(
        [jnp.transpose(a_ref[...]), jnp.transpose(b_ref[...])], axis=1)


_PACK_BM = 2048
_HALF = _PACK_BM * 245


def _tc_pack(table_t):
    nb = _HALF // _PACK_BM
    last = -(-table_t.shape[1] // _PACK_BM) - 1
    return pl.pallas_call(
        _pack_body,
        grid=(nb,),
        in_specs=[
            pl.BlockSpec((_D, _PACK_BM), lambda i: (0, i)),
            pl.BlockSpec((_D, _PACK_BM), lambda i: (0, jnp.minimum(i + nb, last))),
        ],
        out_specs=pl.BlockSpec((_PACK_BM, _DP), lambda i: (i, 0)),
        out_shape=jax.ShapeDtypeStruct((_HALF, _DP), jnp.float32),
    )(table_t, table_t)


def _mlp_body(x_ref, w1_ref, b1_ref, w2_ref, b2_ref, o_ref):
    x = x_ref[...] * (1.0 / _L)
    h = jnp.dot(x, w1_ref[...], preferred_element_type=jnp.float32)
    h = jnp.maximum(h + b1_ref[...], 0.0)
    o = jnp.dot(h, w2_ref[...], preferred_element_type=jnp.float32)
    o_ref[...] = o + b2_ref[...]


def _tc_mlp(pooled, w1t, b1, w2t, b2):
    bm = 2048
    h1 = w1t.shape[1]
    h2 = w2t.shape[1]
    return pl.pallas_call(
        _mlp_body,
        grid=(_B // bm,),
        in_specs=[
            pl.BlockSpec((bm, _D), lambda i: (i, 0)),
            pl.BlockSpec((_D, h1), lambda i: (0, 0)),
            pl.BlockSpec((1, h1), lambda i: (0, 0)),
            pl.BlockSpec((h1, h2), lambda i: (0, 0)),
            pl.BlockSpec((1, h2), lambda i: (0, 0)),
        ],
        out_specs=pl.BlockSpec((bm, h2), lambda i: (i, 0)),
        out_shape=jax.ShapeDtypeStruct((_B, h2), jnp.float32),
    )(pooled, w1t, b1, w2t, b2)


def kernel(arg0_1, arg1_1, arg2_1, arg3_1, arg4_1, arg5_1):
    table2 = _tc_pack(arg0_1.T)
    idx_flat = arg1_1.reshape(-1)
    hi = idx_flat >= _HALF
    idx_pair = jnp.where(hi, idx_flat - _HALF, idx_flat)
    off_col = hi.astype(jnp.int32) * _D
    pooled = _sc_pool(table2, idx_pair, off_col)
    w1t = arg2_1.T
    w2t = arg4_1.T
    b1 = arg3_1.reshape(1, -1)
    b2 = arg5_1.reshape(1, -1)
    out = _tc_mlp(pooled, w1t, b1, w2t, b2)
    return (out,)


# speedup vs baseline: 1.9421x; 1.1316x over previous
import functools

import jax
import jax.numpy as jnp
from jax import lax
from jax.experimental import pallas as pl
from jax.experimental.pallas import tpu as pltpu
from jax.experimental.pallas import tpu_sc as plsc

_NUM_CORES = 2
_NUM_SUBCORES = 16
_NUM_WORKERS = _NUM_CORES * _NUM_SUBCORES
_LANES = 16

_B = 16384
_L = 50
_D = 64
_DP = 2 * _D
_DV = _D // _LANES

_SAMPLES_PER_W = _B // _NUM_WORKERS
_CHUNK_SAMPLES = 8
_CHUNK_ROWS = _CHUNK_SAMPLES * _L
_GATHER_ROWS = 80
_GATHERS_PER_CHUNK = _CHUNK_ROWS // _GATHER_ROWS
_CHUNKS_PER_W = _SAMPLES_PER_W // _CHUNK_SAMPLES


def _sc_pool(table2, idx_pair, off_col):
    mesh = plsc.VectorSubcoreMesh(
        core_axis_name="c", subcore_axis_name="s",
        num_cores=_NUM_CORES, num_subcores=_NUM_SUBCORES)

    nbuf = 2

    @functools.partial(
        pl.kernel,
        out_type=jax.ShapeDtypeStruct((_B, _D), jnp.float32),
        mesh=mesh,
        scratch_types=[
            pltpu.VMEM((nbuf * _CHUNK_ROWS,), jnp.int32),
            pltpu.VMEM((nbuf * (_CHUNK_ROWS + _LANES),), jnp.int32),
            pltpu.VMEM((nbuf, _CHUNK_ROWS, _DP), jnp.float32),
            pltpu.VMEM((nbuf * _CHUNK_SAMPLES, _D), jnp.float32),
            pltpu.SemaphoreType.DMA,
            pltpu.SemaphoreType.DMA,
            pltpu.SemaphoreType.DMA,
            pltpu.SemaphoreType.DMA,
            pltpu.SemaphoreType.DMA,
            pltpu.SemaphoreType.DMA,
        ],
        compiler_params=pltpu.CompilerParams(use_tc_tiling_on_sc=True),
    )
    def sc_kernel(table_hbm, idx_hbm, off_hbm, out_hbm,
                  idx_v, off_v, rows_v, pooled_v,
                  semi0, semi1, semg0, semg1, semo0, semo1):
        semi = (semi0, semi1)
        semg = (semg0, semg1)
        semo = (semo0, semo1)
        wid = lax.axis_index("s") * _NUM_CORES + lax.axis_index("c")
        w_row0 = wid * (_SAMPLES_PER_W * _L)
        w_samp0 = wid * _SAMPLES_PER_W

        def idx_copies(c, b):
            base = w_row0 + c * _CHUNK_ROWS
            return (
                pltpu.make_async_copy(
                    idx_hbm.at[pl.ds(base, _CHUNK_ROWS)],
                    idx_v.at[pl.ds(b * _CHUNK_ROWS, _CHUNK_ROWS)], semi[b]),
                pltpu.make_async_copy(
                    off_hbm.at[pl.ds(base, _CHUNK_ROWS)],
                    off_v.at[pl.ds(b * (_CHUNK_ROWS + _LANES), _CHUNK_ROWS)],
                    semi[b]),
            )

        def gather_copies(b):
            out = []
            for g in range(_GATHERS_PER_CHUNK):
                sl = pl.ds(g * _GATHER_ROWS, _GATHER_ROWS)
                out.append(pltpu.make_async_copy(
                    table_hbm.at[idx_v.at[pl.ds(
                        b * _CHUNK_ROWS + g * _GATHER_ROWS, _GATHER_ROWS)]],
                    rows_v.at[b].at[sl], semg[b]))
            return out

        def out_copy(c, b):
            return pltpu.make_async_copy(
                pooled_v.at[pl.ds(b * _CHUNK_SAMPLES, _CHUNK_SAMPLES)],
                out_hbm.at[pl.ds(w_samp0 + c * _CHUNK_SAMPLES, _CHUNK_SAMPLES)],
                semo[b])

        def accumulate(b):
            def sample_body(s, carry2):
                accs = [jnp.zeros((_LANES,), jnp.float32) for _ in range(_DV)]
                base = s * _L
                ob = b * (_CHUNK_ROWS + _LANES)
                off_vecs = [off_v[pl.ds(ob + base + m * _LANES, _LANES)]
                            for m in range(-(-_L // _LANES))]
                for j in range(_L):
                    r = base + j
                    o = off_vecs[j // _LANES][j % _LANES]
                    for k in range(_DV):
                        accs[k] = accs[k] + rows_v[b, r, pl.ds(o + k * _LANES, _LANES)]
                for k in range(_DV):
                    pooled_v[b * _CHUNK_SAMPLES + s,
                             pl.ds(k * _LANES, _LANES)] = accs[k]
                return carry2

            lax.fori_loop(0, _CHUNK_SAMPLES, sample_body, 0)

        for cp in idx_copies(0, 0):
            cp.start()
        for cp in idx_copies(1, 1):
            cp.start()
        for cp in idx_copies(0, 0):
            cp.wait()
        for cp in gather_copies(0):
            cp.start()

        def pair_body(it, carry):
            for b in range(nbuf):
                c = it * nbuf + b
                nb = 1 - b

                @pl.when(c < _CHUNKS_PER_W - 1)
                def _():
                    for cp in idx_copies(c + 1, nb):
                        cp.wait()
                    for cp in gather_copies(nb):
                        cp.start()

                for cp in gather_copies(b):
                    cp.wait()

                @pl.when(c >= 2)
                def _():
                    out_copy(c - 2, b).wait()

                accumulate(b)

                @pl.when(c + 2 < _CHUNKS_PER_W)
                def _():
                    for cp in idx_copies(c + 2, b):
                        cp.start()

                out_copy(c, b).start()
            return carry

        lax.fori_loop(0, _CHUNKS_PER_W // nbuf, pair_body, 0)
        out_copy(_CHUNKS_PER_W - 2, 0).wait()
        out_copy(_CHUNKS_PER_W - 1, 1).wait()

    return sc_kernel(table2, idx_pair, off_col)


def _pack_body(a_ref, b_ref, o_ref):
    o_ref[...] = jnp.concatenate(
        [jnp.transpose(a_ref[...]), jnp.transpose(b_ref[...])], axis=1)


_PACK_BM = 4096
_HALF = _PACK_BM * 123


def _tc_pack(table_t):
    nb = _HALF // _PACK_BM
    last = -(-table_t.shape[1] // _PACK_BM) - 1
    return pl.pallas_call(
        _pack_body,
        grid=(nb,),
        in_specs=[
            pl.BlockSpec((_D, _PACK_BM), lambda i: (0, i)),
            pl.BlockSpec((_D, _PACK_BM), lambda i: (0, jnp.minimum(i + nb, last))),
        ],
        out_specs=pl.BlockSpec((_PACK_BM, _DP), lambda i: (i, 0)),
        out_shape=jax.ShapeDtypeStruct((_HALF, _DP), jnp.float32),
    )(table_t, table_t)


def _mlp_body(x_ref, w1_ref, b1_ref, w2_ref, b2_ref, o_ref):
    x = x_ref[...] * (1.0 / _L)
    h = jnp.dot(x, w1_ref[...], preferred_element_type=jnp.float32)
    h = jnp.maximum(h + b1_ref[...], 0.0)
    o = jnp.dot(h, w2_ref[...], preferred_element_type=jnp.float32)
    o_ref[...] = o + b2_ref[...]


def _tc_mlp(pooled, w1t, b1, w2t, b2):
    bm = 2048
    h1 = w1t.shape[1]
    h2 = w2t.shape[1]
    return pl.pallas_call(
        _mlp_body,
        grid=(_B // bm,),
        in_specs=[
            pl.BlockSpec((bm, _D), lambda i: (i, 0)),
            pl.BlockSpec((_D, h1), lambda i: (0, 0)),
            pl.BlockSpec((1, h1), lambda i: (0, 0)),
            pl.BlockSpec((h1, h2), lambda i: (0, 0)),
            pl.BlockSpec((1, h2), lambda i: (0, 0)),
        ],
        out_specs=pl.BlockSpec((bm, h2), lambda i: (i, 0)),
        out_shape=jax.ShapeDtypeStruct((_B, h2), jnp.float32),
    )(pooled, w1t, b1, w2t, b2)


def kernel(arg0_1, arg1_1, arg2_1, arg3_1, arg4_1, arg5_1):
    table2 = _tc_pack(arg0_1.T)
    idx_flat = arg1_1.reshape(-1)
    hi = idx_flat >= _HALF
    idx_pair = jnp.where(hi, idx_flat - _HALF, idx_flat)
    off_col = hi.astype(jnp.int32) * _D
    pooled = _sc_pool(table2, idx_pair, off_col)
    w1t = arg2_1.T
    w2t = arg4_1.T
    b1 = arg3_1.reshape(1, -1)
    b2 = arg5_1.reshape(1, -1)
    out = _tc_mlp(pooled, w1t, b1, w2t, b2)
    return (out,)


# speedup vs baseline: 2.0830x; 1.0726x over previous
import functools

import jax
import jax.numpy as jnp
from jax import lax
from jax.experimental import pallas as pl
from jax.experimental.pallas import tpu as pltpu
from jax.experimental.pallas import tpu_sc as plsc

_NUM_CORES = 2
_NUM_SUBCORES = 16
_NUM_WORKERS = _NUM_CORES * _NUM_SUBCORES
_LANES = 16

_B = 16384
_L = 50
_D = 64
_DP = 2 * _D
_DV = _D // _LANES

_SAMPLES_PER_W = _B // _NUM_WORKERS
_CHUNK_SAMPLES = 8
_CHUNK_ROWS = _CHUNK_SAMPLES * _L
_GATHER_ROWS = 80
_GATHERS_PER_CHUNK = _CHUNK_ROWS // _GATHER_ROWS
_CHUNKS_PER_W = _SAMPLES_PER_W // _CHUNK_SAMPLES


def _sc_pool(table2, idx_pair, off_col):
    mesh = plsc.VectorSubcoreMesh(
        core_axis_name="c", subcore_axis_name="s",
        num_cores=_NUM_CORES, num_subcores=_NUM_SUBCORES)

    nbuf = 2

    @functools.partial(
        pl.kernel,
        out_type=jax.ShapeDtypeStruct((_B, _D), jnp.float32),
        mesh=mesh,
        scratch_types=[
            pltpu.VMEM((nbuf * _CHUNK_ROWS,), jnp.int32),
            pltpu.VMEM((nbuf * (_CHUNK_ROWS + _LANES),), jnp.int32),
            pltpu.VMEM((nbuf, _CHUNK_ROWS, _DP), jnp.float32),
            pltpu.VMEM((nbuf * _CHUNK_SAMPLES, _D), jnp.float32),
            pltpu.SemaphoreType.DMA,
            pltpu.SemaphoreType.DMA,
            pltpu.SemaphoreType.DMA,
            pltpu.SemaphoreType.DMA,
            pltpu.SemaphoreType.DMA,
            pltpu.SemaphoreType.DMA,
        ],
        compiler_params=pltpu.CompilerParams(use_tc_tiling_on_sc=True),
    )
    def sc_kernel(table_hbm, idx_hbm, off_hbm, out_hbm,
                  idx_v, off_v, rows_v, pooled_v,
                  semi0, semi1, semg0, semg1, semo0, semo1):
        semi = (semi0, semi1)
        semg = (semg0, semg1)
        semo = (semo0, semo1)
        wid = lax.axis_index("s") * _NUM_CORES + lax.axis_index("c")
        w_row0 = wid * (_SAMPLES_PER_W * _L)
        w_samp0 = wid * _SAMPLES_PER_W

        def idx_copies(c, b):
            base = w_row0 + c * _CHUNK_ROWS
            return (
                pltpu.make_async_copy(
                    idx_hbm.at[pl.ds(base, _CHUNK_ROWS)],
                    idx_v.at[pl.ds(b * _CHUNK_ROWS, _CHUNK_ROWS)], semi[b]),
                pltpu.make_async_copy(
                    off_hbm.at[pl.ds(base, _CHUNK_ROWS)],
                    off_v.at[pl.ds(b * (_CHUNK_ROWS + _LANES), _CHUNK_ROWS)],
                    semi[b]),
            )

        def gather_copies(b):
            out = []
            for g in range(_GATHERS_PER_CHUNK):
                sl = pl.ds(g * _GATHER_ROWS, _GATHER_ROWS)
                out.append(pltpu.make_async_copy(
                    table_hbm.at[idx_v.at[pl.ds(
                        b * _CHUNK_ROWS + g * _GATHER_ROWS, _GATHER_ROWS)]],
                    rows_v.at[b].at[sl], semg[b]))
            return out

        def out_copy(c, b):
            return pltpu.make_async_copy(
                pooled_v.at[pl.ds(b * _CHUNK_SAMPLES, _CHUNK_SAMPLES)],
                out_hbm.at[pl.ds(w_samp0 + c * _CHUNK_SAMPLES, _CHUNK_SAMPLES)],
                semo[b])

        def accumulate(b):
            def sample_body(s, carry2):
                accs = [jnp.zeros((_LANES,), jnp.float32) for _ in range(_DV)]
                base = s * _L
                ob = b * (_CHUNK_ROWS + _LANES)
                off_vecs = [off_v[pl.ds(ob + base + m * _LANES, _LANES)]
                            for m in range(-(-_L // _LANES))]
                for j in range(_L):
                    r = base + j
                    o = off_vecs[j // _LANES][j % _LANES]
                    for k in range(_DV):
                        accs[k] = accs[k] + rows_v[b, r, pl.ds(o + k * _LANES, _LANES)]
                for k in range(_DV):
                    pooled_v[b * _CHUNK_SAMPLES + s,
                             pl.ds(k * _LANES, _LANES)] = accs[k]
                return carry2

            lax.fori_loop(0, _CHUNK_SAMPLES, sample_body, 0)

        for cp in idx_copies(0, 0):
            cp.start()
        for cp in idx_copies(1, 1):
            cp.start()
        for cp in idx_copies(0, 0):
            cp.wait()
        for cp in gather_copies(0):
            cp.start()

        def pair_body(it, carry):
            for b in range(nbuf):
                c = it * nbuf + b
                nb = 1 - b

                @pl.when(c < _CHUNKS_PER_W - 1)
                def _():
                    for cp in idx_copies(c + 1, nb):
                        cp.wait()
                    for cp in gather_copies(nb):
                        cp.start()

                for cp in gather_copies(b):
                    cp.wait()

                @pl.when(c >= 2)
                def _():
                    out_copy(c - 2, b).wait()

                accumulate(b)

                @pl.when(c + 2 < _CHUNKS_PER_W)
                def _():
                    for cp in idx_copies(c + 2, b):
                        cp.start()

                out_copy(c, b).start()
            return carry

        lax.fori_loop(0, _CHUNKS_PER_W // nbuf, pair_body, 0)
        out_copy(_CHUNKS_PER_W - 2, 0).wait()
        out_copy(_CHUNKS_PER_W - 1, 1).wait()

    return sc_kernel(table2, idx_pair, off_col)


def _pack_body(a_ref, b_ref, o_ref):
    o_ref[...] = jnp.concatenate(
        [jnp.transpose(a_ref[...]), jnp.transpose(b_ref[...])], axis=1)


_PACK_BM = 8192
_HALF = _PACK_BM * 62


def _tc_pack(table_t):
    nb = _HALF // _PACK_BM
    last = -(-table_t.shape[1] // _PACK_BM) - 1
    return pl.pallas_call(
        _pack_body,
        grid=(nb,),
        in_specs=[
            pl.BlockSpec((_D, _PACK_BM), lambda i: (0, i)),
            pl.BlockSpec((_D, _PACK_BM), lambda i: (0, jnp.minimum(i + nb, last))),
        ],
        out_specs=pl.BlockSpec((_PACK_BM, _DP), lambda i: (i, 0)),
        out_shape=jax.ShapeDtypeStruct((_HALF, _DP), jnp.float32),
    )(table_t, table_t)


def _mlp_body(x_ref, w1_ref, b1_ref, w2_ref, b2_ref, o_ref):
    x = x_ref[...] * (1.0 / _L)
    h = jnp.dot(x, w1_ref[...], preferred_element_type=jnp.float32)
    h = jnp.maximum(h + b1_ref[...], 0.0)
    o = jnp.dot(h, w2_ref[...], preferred_element_type=jnp.float32)
    o_ref[...] = o + b2_ref[...]


def _tc_mlp(pooled, w1t, b1, w2t, b2):
    bm = 2048
    h1 = w1t.shape[1]
    h2 = w2t.shape[1]
    return pl.pallas_call(
        _mlp_body,
        grid=(_B // bm,),
        in_specs=[
            pl.BlockSpec((bm, _D), lambda i: (i, 0)),
            pl.BlockSpec((_D, h1), lambda i: (0, 0)),
            pl.BlockSpec((1, h1), lambda i: (0, 0)),
            pl.BlockSpec((h1, h2), lambda i: (0, 0)),
            pl.BlockSpec((1, h2), lambda i: (0, 0)),
        ],
        out_specs=pl.BlockSpec((bm, h2), lambda i: (i, 0)),
        out_shape=jax.ShapeDtypeStruct((_B, h2), jnp.float32),
    )(pooled, w1t, b1, w2t, b2)


def kernel(arg0_1, arg1_1, arg2_1, arg3_1, arg4_1, arg5_1):
    table2 = _tc_pack(arg0_1.T)
    idx_flat = arg1_1.reshape(-1)
    hi = idx_flat >= _HALF
    idx_pair = jnp.where(hi, idx_flat - _HALF, idx_flat)
    off_col = hi.astype(jnp.int32) * _D
    pooled = _sc_pool(table2, idx_pair, off_col)
    w1t = arg2_1.T
    w2t = arg4_1.T
    b1 = arg3_1.reshape(1, -1)
    b2 = arg5_1.reshape(1, -1)
    out = _tc_mlp(pooled, w1t, b1, w2t, b2)
    return (out,)


# speedup vs baseline: 2.1443x; 1.0294x over previous
import functools

import jax
import jax.numpy as jnp
from jax import lax
from jax.experimental import pallas as pl
from jax.experimental.pallas import tpu as pltpu
from jax.experimental.pallas import tpu_sc as plsc

_NUM_CORES = 2
_NUM_SUBCORES = 16
_NUM_WORKERS = _NUM_CORES * _NUM_SUBCORES
_LANES = 16

_B = 16384
_L = 50
_D = 64
_DP = 2 * _D
_DV = _D // _LANES

_SAMPLES_PER_W = _B // _NUM_WORKERS
_CHUNK_SAMPLES = 8
_CHUNK_ROWS = _CHUNK_SAMPLES * _L
_GATHER_ROWS = 80
_GATHERS_PER_CHUNK = _CHUNK_ROWS // _GATHER_ROWS
_CHUNKS_PER_W = _SAMPLES_PER_W // _CHUNK_SAMPLES


def _sc_pool(table2, idx_pair, off_col):
    mesh = plsc.VectorSubcoreMesh(
        core_axis_name="c", subcore_axis_name="s",
        num_cores=_NUM_CORES, num_subcores=_NUM_SUBCORES)

    nbuf = 2

    @functools.partial(
        pl.kernel,
        out_type=jax.ShapeDtypeStruct((_B, _D), jnp.float32),
        mesh=mesh,
        scratch_types=[
            pltpu.VMEM((nbuf * _CHUNK_ROWS,), jnp.int32),
            pltpu.VMEM((nbuf * (_CHUNK_ROWS + _LANES),), jnp.int32),
            pltpu.VMEM((nbuf, _CHUNK_ROWS, _DP), jnp.float32),
            pltpu.VMEM((nbuf * _CHUNK_SAMPLES, _D), jnp.float32),
            pltpu.SemaphoreType.DMA,
            pltpu.SemaphoreType.DMA,
            pltpu.SemaphoreType.DMA,
            pltpu.SemaphoreType.DMA,
            pltpu.SemaphoreType.DMA,
            pltpu.SemaphoreType.DMA,
        ],
        compiler_params=pltpu.CompilerParams(use_tc_tiling_on_sc=True),
    )
    def sc_kernel(table_hbm, idx_hbm, off_hbm, out_hbm,
                  idx_v, off_v, rows_v, pooled_v,
                  semi0, semi1, semg0, semg1, semo0, semo1):
        semi = (semi0, semi1)
        semg = (semg0, semg1)
        semo = (semo0, semo1)
        wid = lax.axis_index("s") * _NUM_CORES + lax.axis_index("c")
        w_row0 = wid * (_SAMPLES_PER_W * _L)
        w_samp0 = wid * _SAMPLES_PER_W

        def idx_copies(c, b):
            base = w_row0 + c * _CHUNK_ROWS
            return (
                pltpu.make_async_copy(
                    idx_hbm.at[pl.ds(base, _CHUNK_ROWS)],
                    idx_v.at[pl.ds(b * _CHUNK_ROWS, _CHUNK_ROWS)], semi[b]),
                pltpu.make_async_copy(
                    off_hbm.at[pl.ds(base, _CHUNK_ROWS)],
                    off_v.at[pl.ds(b * (_CHUNK_ROWS + _LANES), _CHUNK_ROWS)],
                    semi[b]),
            )

        def gather_copies(b):
            out = []
            for g in range(_GATHERS_PER_CHUNK):
                sl = pl.ds(g * _GATHER_ROWS, _GATHER_ROWS)
                out.append(pltpu.make_async_copy(
                    table_hbm.at[idx_v.at[pl.ds(
                        b * _CHUNK_ROWS + g * _GATHER_ROWS, _GATHER_ROWS)]],
                    rows_v.at[b].at[sl], semg[b]))
            return out

        def out_copy(c, b):
            return pltpu.make_async_copy(
                pooled_v.at[pl.ds(b * _CHUNK_SAMPLES, _CHUNK_SAMPLES)],
                out_hbm.at[pl.ds(w_samp0 + c * _CHUNK_SAMPLES, _CHUNK_SAMPLES)],
                semo[b])

        def accumulate(b):
            def sample_body(s, carry2):
                accs = [jnp.zeros((_LANES,), jnp.float32) for _ in range(_DV)]
                base = s * _L
                ob = b * (_CHUNK_ROWS + _LANES)
                off_vecs = [off_v[pl.ds(ob + base + m * _LANES, _LANES)]
                            for m in range(-(-_L // _LANES))]
                for j in range(_L):
                    r = base + j
                    o = off_vecs[j // _LANES][j % _LANES]
                    for k in range(_DV):
                        accs[k] = accs[k] + rows_v[b, r, pl.ds(o + k * _LANES, _LANES)]
                for k in range(_DV):
                    pooled_v[b * _CHUNK_SAMPLES + s,
                             pl.ds(k * _LANES, _LANES)] = accs[k]
                return carry2

            lax.fori_loop(0, _CHUNK_SAMPLES, sample_body, 0)

        for cp in idx_copies(0, 0):
            cp.start()
        for cp in idx_copies(1, 1):
            cp.start()
        for cp in idx_copies(0, 0):
            cp.wait()
        for cp in gather_copies(0):
            cp.start()

        def pair_body(it, carry):
            for b in range(nbuf):
                c = it * nbuf + b
                nb = 1 - b

                @pl.when(c < _CHUNKS_PER_W - 1)
                def _():
                    for cp in idx_copies(c + 1, nb):
                        cp.wait()
                    for cp in gather_copies(nb):
                        cp.start()

                for cp in gather_copies(b):
                    cp.wait()

                @pl.when(c >= 2)
                def _():
                    out_copy(c - 2, b).wait()

                accumulate(b)

                @pl.when(c + 2 < _CHUNKS_PER_W)
                def _():
                    for cp in idx_copies(c + 2, b):
                        cp.start()

                out_copy(c, b).start()
            return carry

        lax.fori_loop(0, _CHUNKS_PER_W // nbuf, pair_body, 0)
        out_copy(_CHUNKS_PER_W - 2, 0).wait()
        out_copy(_CHUNKS_PER_W - 1, 1).wait()

    return sc_kernel(table2, idx_pair, off_col)


def _pack_body(a_ref, b_ref, o_ref):
    o_ref[...] = jnp.concatenate(
        [jnp.transpose(a_ref[...]), jnp.transpose(b_ref[...])], axis=1)


_PACK_BM = 16384
_HALF = _PACK_BM * 31


def _tc_pack(table_t):
    nb = _HALF // _PACK_BM
    last = -(-table_t.shape[1] // _PACK_BM) - 1
    return pl.pallas_call(
        _pack_body,
        grid=(nb,),
        in_specs=[
            pl.BlockSpec((_D, _PACK_BM), lambda i: (0, i)),
            pl.BlockSpec((_D, _PACK_BM), lambda i: (0, jnp.minimum(i + nb, last))),
        ],
        out_specs=pl.BlockSpec((_PACK_BM, _DP), lambda i: (i, 0)),
        out_shape=jax.ShapeDtypeStruct((_HALF, _DP), jnp.float32),
    )(table_t, table_t)


def _mlp_body(x_ref, w1_ref, b1_ref, w2_ref, b2_ref, o_ref):
    x = x_ref[...] * (1.0 / _L)
    h = jnp.dot(x, w1_ref[...], preferred_element_type=jnp.float32)
    h = jnp.maximum(h + b1_ref[...], 0.0)
    o = jnp.dot(h, w2_ref[...], preferred_element_type=jnp.float32)
    o_ref[...] = o + b2_ref[...]


def _tc_mlp(pooled, w1t, b1, w2t, b2):
    bm = 2048
    h1 = w1t.shape[1]
    h2 = w2t.shape[1]
    return pl.pallas_call(
        _mlp_body,
        grid=(_B // bm,),
        in_specs=[
            pl.BlockSpec((bm, _D), lambda i: (i, 0)),
            pl.BlockSpec((_D, h1), lambda i: (0, 0)),
            pl.BlockSpec((1, h1), lambda i: (0, 0)),
            pl.BlockSpec((h1, h2), lambda i: (0, 0)),
            pl.BlockSpec((1, h2), lambda i: (0, 0)),
        ],
        out_specs=pl.BlockSpec((bm, h2), lambda i: (i, 0)),
        out_shape=jax.ShapeDtypeStruct((_B, h2), jnp.float32),
    )(pooled, w1t, b1, w2t, b2)


def kernel(arg0_1, arg1_1, arg2_1, arg3_1, arg4_1, arg5_1):
    table2 = _tc_pack(arg0_1.T)
    idx_flat = arg1_1.reshape(-1)
    hi = idx_flat >= _HALF
    idx_pair = jnp.where(hi, idx_flat - _HALF, idx_flat)
    off_col = hi.astype(jnp.int32) * _D
    pooled = _sc_pool(table2, idx_pair, off_col)
    w1t = arg2_1.T
    w2t = arg4_1.T
    b1 = arg3_1.reshape(1, -1)
    b2 = arg5_1.reshape(1, -1)
    out = _tc_mlp(pooled, w1t, b1, w2t, b2)
    return (out,)
